# baseline (speedup 1.0000x reference)
import functools

import jax
import jax.numpy as jnp
from jax import lax
from jax.experimental import pallas as pl


def _sum_lanes_i16(a):
    n = a.shape[1]
    while n > 128:
        n //= 2
        a = a[:, :n] + a[:, n:]
    return jnp.sum(a.astype(jnp.int32), axis=-1, keepdims=True)


def _greedy(data, kth, limit, nbits, bias):
    rows = data.shape[0]
    kth32 = jnp.int32(kth)

    def step(i, carry):
        p, c_acc = carry
        bit = jnp.left_shift(jnp.int32(1), jnp.int32(nbits - 1) - i)
        cand = jnp.bitwise_or(p, bit)
        trial = (cand - bias).astype(jnp.int16)
        c = _sum_lanes_i16((data < trial).astype(jnp.int16))
        accept = c + limit <= kth32
        return jnp.where(accept, cand, p), jnp.where(accept, c, c_acc)

    p0 = jnp.zeros((rows, 1), jnp.int32)
    return lax.fori_loop(0, nbits, step, (p0, p0))


def _median_body(x_ref, val_ref, idx_ref, *, kth):
    i32min = jnp.int32(-(2 ** 31))
    i16max = jnp.int16(2 ** 15 - 1)
    xb = x_ref[...]
    rows, n = xb.shape
    s = lax.bitcast_convert_type(xb, jnp.int32)
    key = jnp.where(s >= 0, s, jnp.bitwise_xor(jnp.bitwise_not(s), i32min))
    hi = jnp.right_shift(key, 16).astype(jnp.int16)
    lo = (jnp.bitwise_and(key, 0xFFFF) - 32768).astype(jnp.int16)

    hp, c1 = _greedy(hi, kth, jnp.zeros((rows, 1), jnp.int32), 16, 32768)
    hp16 = (hp - 32768).astype(jnp.int16)

    m1 = (hi == hp16)
    lox = jnp.where(m1, lo, i16max)
    lp, c2 = _greedy(lox, kth, c1, 16, 32768)
    lp16 = (lp - 32768).astype(jnp.int16)

    iota = lax.broadcasted_iota(jnp.int16, (rows, n), 1)
    iox = jnp.where(jnp.logical_and(m1, lo == lp16), iota, i16max)
    med_idx, _ = _greedy(iox, kth, c1 + c2, 11, 0)

    v = jnp.bitwise_or(jnp.left_shift(hp - 32768, 16), lp)
    sv = jnp.where(v >= 0, v, jnp.bitwise_not(jnp.bitwise_xor(v, i32min)))
    val_ref[...] = lax.bitcast_convert_type(sv, jnp.float32)
    idx_ref[...] = med_idx


def _median_2d(x2, block_rows):
    m, n = x2.shape
    kth = (n - 1) // 2
    grid = (m // block_rows,)
    vals, idx = pl.pallas_call(
        functools.partial(_median_body, kth=kth),
        grid=grid,
        in_specs=[pl.BlockSpec((block_rows, n), lambda j: (j, 0))],
        out_specs=[
            pl.BlockSpec((block_rows, 1), lambda j: (j, 0)),
            pl.BlockSpec((block_rows, 1), lambda j: (j, 0)),
        ],
        out_shape=[
            jax.ShapeDtypeStruct((m, 1), jnp.float32),
            jax.ShapeDtypeStruct((m, 1), jnp.int32),
        ],
    )(x2)
    return vals[:, 0], idx[:, 0]


def kernel(x):
    b, s, n = x.shape
    m = b * s
    x2 = x.reshape(m, n)
    block_rows = 256 if m % 256 == 0 else m
    vals, idx = _median_2d(x2, block_rows)
    return vals.reshape(b, s), idx.reshape(b, s).astype(jnp.int64)


# speedup vs baseline: 1.6584x; 1.6584x over previous
import functools

import jax
import jax.numpy as jnp
from jax import lax
from jax.experimental import pallas as pl


def _sum_lanes_i16(a):
    n = a.shape[1]
    while n > 128:
        n //= 2
        a = a[:, :n] + a[:, n:]
    return jnp.sum(a.astype(jnp.int32), axis=-1, keepdims=True)


def _greedy_multi(datas, kth, limits, nbits, bias):
    kth32 = jnp.int32(kth)
    ps = [jnp.zeros((d.shape[0], 1), jnp.int32) for d in datas]
    cs = [jnp.zeros((d.shape[0], 1), jnp.int32) for d in datas]
    for i in range(nbits):
        bit = 1 << (nbits - 1 - i)
        for g, data in enumerate(datas):
            cand = jnp.bitwise_or(ps[g], bit)
            trial = (cand - bias).astype(jnp.int16)
            c = _sum_lanes_i16((data < trial).astype(jnp.int16))
            accept = c + limits[g] <= kth32
            ps[g] = jnp.where(accept, cand, ps[g])
            cs[g] = jnp.where(accept, c, cs[g])
    return ps, cs


def _median_body(x_ref, val_ref, idx_ref, *, kth, groups):
    i32min = jnp.int32(-(2 ** 31))
    i16max = jnp.int16(2 ** 15 - 1)
    xb = x_ref[...]
    rows, n = xb.shape
    gr = rows // groups
    s = lax.bitcast_convert_type(xb, jnp.int32)
    key = jnp.where(s >= 0, s, jnp.bitwise_xor(jnp.bitwise_not(s), i32min))
    hi = jnp.right_shift(key, 16).astype(jnp.int16)
    lo = (jnp.bitwise_and(key, 0xFFFF) - 32768).astype(jnp.int16)

    his = [hi[g * gr:(g + 1) * gr] for g in range(groups)]
    los = [lo[g * gr:(g + 1) * gr] for g in range(groups)]
    zeros = [jnp.zeros((gr, 1), jnp.int32) for _ in range(groups)]

    hps, c1s = _greedy_multi(his, kth, zeros, 16, 32768)

    m1s = [h == (hp - 32768).astype(jnp.int16) for h, hp in zip(his, hps)]
    loxs = [jnp.where(m, l, i16max) for m, l in zip(m1s, los)]
    lps, c2s = _greedy_multi(loxs, kth, c1s, 16, 32768)

    iota = lax.broadcasted_iota(jnp.int16, (gr, n), 1)
    ioxs = [jnp.where(jnp.logical_and(m, l == (lp - 32768).astype(jnp.int16)),
                      iota, i16max)
            for m, l, lp in zip(m1s, los, lps)]
    limits3 = [c1 + c2 for c1, c2 in zip(c1s, c2s)]
    idxs, _ = _greedy_multi(ioxs, kth, limits3, 11, 0)

    for g in range(groups):
        v = jnp.bitwise_or(jnp.left_shift(hps[g] - 32768, 16), lps[g])
        sv = jnp.where(v >= 0, v,
                       jnp.bitwise_not(jnp.bitwise_xor(v, i32min)))
        val_ref[g * gr:(g + 1) * gr, :] = lax.bitcast_convert_type(
            sv, jnp.float32)
        idx_ref[g * gr:(g + 1) * gr, :] = idxs[g]


def _median_2d(x2, block_rows):
    m, n = x2.shape
    kth = (n - 1) // 2
    grid = (m // block_rows,)
    vals, idx = pl.pallas_call(
        functools.partial(_median_body, kth=kth, groups=2),
        grid=grid,
        in_specs=[pl.BlockSpec((block_rows, n), lambda j: (j, 0))],
        out_specs=[
            pl.BlockSpec((block_rows, 1), lambda j: (j, 0)),
            pl.BlockSpec((block_rows, 1), lambda j: (j, 0)),
        ],
        out_shape=[
            jax.ShapeDtypeStruct((m, 1), jnp.float32),
            jax.ShapeDtypeStruct((m, 1), jnp.int32),
        ],
    )(x2)
    return vals[:, 0], idx[:, 0]


def kernel(x):
    b, s, n = x.shape
    m = b * s
    x2 = x.reshape(m, n)
    block_rows = 256 if m % 256 == 0 else m
    vals, idx = _median_2d(x2, block_rows)
    return vals.reshape(b, s), idx.reshape(b, s).astype(jnp.int64)
